# baseline (speedup 1.0000x reference)
import jax
import jax.numpy as jnp
from jax import lax
from jax.experimental import pallas as pl
from jax.experimental.pallas import tpu as pltpu
from jax.experimental.pallas import tpu_sc as plsc

B = 4096
G = 4
L = 50
D = 128
V = 10000
NW = 32
ROWS_PER_W = B // NW


def _tc_project(b_ref, w_ref, table_ref, out_ref):
    out_ref[...] = lax.dot_general(
        w_ref[...], table_ref[...],
        (((1,), (1,)), ((), ())),
        preferred_element_type=jnp.float32,
    ) * (1.0 / L) + b_ref[0] * (1.0 / (G * L))


def _sc_pool(x_hbm, t2_hbm, out_hbm, idx_v, t2_v, lvt_v, out_v,
             sem_i0, sem_i1, sem_t0, sem_t1, sem_t2, sem_t3):
    wid = lax.axis_index("s") * 2 + lax.axis_index("c")
    base_row = wid * ROWS_PER_W
    HALF = ROWS_PER_W // 2
    sems_t = [sem_t0, sem_t1, sem_t2, sem_t3]
    cp_t = [pltpu.async_copy(t2_hbm.at[g], t2_v.at[g], sems_t[g])
            for g in range(G)]
    cp_a = pltpu.async_copy(x_hbm.at[pl.ds(base_row, HALF)],
                            idx_v.at[pl.ds(0, HALF)], sem_i0)
    cp_b = pltpu.async_copy(x_hbm.at[pl.ds(base_row + HALF, HALF)],
                            idx_v.at[pl.ds(HALF, HALF)], sem_i1)
    lanes = lax.iota(jnp.int32, 16)
    for l0 in range(L):
        lvt_v[pl.ds(16 * l0, 16)] = lax.rem(lanes + l0, L)

    for g in range(G):
        cp_t[g].wait()
        gv = jnp.full((16,), g, jnp.int32)
        for half in range(2):
            if g == 0:
                (cp_a if half == 0 else cp_b).wait()

            def rg_body(rg, _, gv=gv, g=g):
                rv = lanes + rg * 16
                acc = jnp.zeros((16,), jnp.float32)
                for l0 in range(L):
                    lv = lvt_v[pl.ds(16 * l0, 16)]
                    iv = plsc.load_gather(idx_v, [rv, gv, lv])
                    acc = acc + plsc.load_gather(t2_v, [gv, iv])
                if g == 0:
                    out_v[pl.ds(rg * 16, 16)] = acc
                else:
                    out_v[pl.ds(rg * 16, 16)] = out_v[pl.ds(rg * 16, 16)] + acc
                return 0

            lax.fori_loop(half * 4, half * 4 + 4, rg_body, 0)

    pltpu.sync_copy(out_v, out_hbm.at[pl.ds(base_row, ROWS_PER_W)])


def kernel(x, table, W, b):
    w2 = W.reshape(G, D)
    t2 = pl.pallas_call(
        _tc_project,
        in_specs=[
            pl.BlockSpec(memory_space=pltpu.SMEM),
            pl.BlockSpec(memory_space=pltpu.VMEM),
            pl.BlockSpec(memory_space=pltpu.VMEM),
        ],
        out_shape=jax.ShapeDtypeStruct((G, V), jnp.float32),
    )(b, w2, table)

    sc = pl.kernel(
        _sc_pool,
        out_type=jax.ShapeDtypeStruct((B,), jnp.float32),
        mesh=plsc.VectorSubcoreMesh(core_axis_name="c", subcore_axis_name="s"),
        compiler_params=pltpu.CompilerParams(needs_layout_passes=False,
                                             use_tc_tiling_on_sc=True),
        scratch_types=[
            pltpu.VMEM((ROWS_PER_W, G, L), jnp.int32),
            pltpu.VMEM((G, V), jnp.float32),
            pltpu.VMEM((16 * L,), jnp.int32),
            pltpu.VMEM((ROWS_PER_W,), jnp.float32),
            pltpu.SemaphoreType.DMA,
            pltpu.SemaphoreType.DMA,
            pltpu.SemaphoreType.DMA,
            pltpu.SemaphoreType.DMA,
            pltpu.SemaphoreType.DMA,
            pltpu.SemaphoreType.DMA,
        ],
    )
    out = sc(x.astype(jnp.int32), t2)
    return out.reshape(B, 1)


# speedup vs baseline: 1.1278x; 1.1278x over previous
import jax
import jax.numpy as jnp
from jax import lax
from jax.experimental import pallas as pl
from jax.experimental.pallas import tpu as pltpu
from jax.experimental.pallas import tpu_sc as plsc

B = 4096
G = 4
L = 50
D = 128
V = 10000
NW = 32
ROWS_PER_W = B // NW


def _tc_project(b_ref, w_ref, table_ref, out_ref):
    out_ref[...] = lax.dot_general(
        w_ref[...], table_ref[...],
        (((1,), (1,)), ((), ())),
        preferred_element_type=jnp.float32,
    ) * (1.0 / L) + b_ref[0] * (1.0 / (G * L))


def _sc_pool(x_hbm, t2_hbm, out_hbm, idx_v, t2_v, lvt_v, out_v,
             sem_i0, sem_i1, sem_t0, sem_t1, sem_t2, sem_t3):
    wid = lax.axis_index("s") * 2 + lax.axis_index("c")
    base_row = wid * ROWS_PER_W
    HALF = ROWS_PER_W // 2
    cp_t2 = pltpu.async_copy(t2_hbm, t2_v, sem_t0)
    cp_a = pltpu.async_copy(x_hbm.at[pl.ds(base_row, HALF)],
                            idx_v.at[pl.ds(0, HALF)], sem_i0)
    cp_b = pltpu.async_copy(x_hbm.at[pl.ds(base_row + HALF, HALF)],
                            idx_v.at[pl.ds(HALF, HALF)], sem_i1)
    lanes = lax.iota(jnp.int32, 16)
    for l0 in range(L):
        lvt_v[pl.ds(16 * l0, 16)] = lax.rem(lanes + l0, L)
    cp_t2.wait()

    def rg_body(rg, _):
        rv = lanes + rg * 16
        acc = jnp.zeros((16,), jnp.float32)
        for g in range(G):
            gv = jnp.full((16,), g, jnp.int32)
            for l0 in range(L):
                lv = lvt_v[pl.ds(16 * l0, 16)]
                iv = plsc.load_gather(idx_v, [rv, gv, lv])
                acc = acc + plsc.load_gather(t2_v, [gv, iv])
        out_v[pl.ds(rg * 16, 16)] = acc
        return 0

    cp_a.wait()
    lax.fori_loop(0, 4, rg_body, 0)
    cp_b.wait()
    lax.fori_loop(4, 8, rg_body, 0)
    pltpu.sync_copy(out_v, out_hbm.at[pl.ds(base_row, ROWS_PER_W)])


def kernel(x, table, W, b):
    w2 = W.reshape(G, D)
    t2 = pl.pallas_call(
        _tc_project,
        in_specs=[
            pl.BlockSpec(memory_space=pltpu.SMEM),
            pl.BlockSpec(memory_space=pltpu.VMEM),
            pl.BlockSpec(memory_space=pltpu.VMEM),
        ],
        out_shape=jax.ShapeDtypeStruct((G, V), jnp.float32),
    )(b, w2, table)

    sc = pl.kernel(
        _sc_pool,
        out_type=jax.ShapeDtypeStruct((B,), jnp.float32),
        mesh=plsc.VectorSubcoreMesh(core_axis_name="c", subcore_axis_name="s"),
        compiler_params=pltpu.CompilerParams(needs_layout_passes=False,
                                             use_tc_tiling_on_sc=True),
        scratch_types=[
            pltpu.VMEM((ROWS_PER_W, G, L), jnp.int32),
            pltpu.VMEM((G, V), jnp.float32),
            pltpu.VMEM((16 * L,), jnp.int32),
            pltpu.VMEM((ROWS_PER_W,), jnp.float32),
            pltpu.SemaphoreType.DMA,
            pltpu.SemaphoreType.DMA,
            pltpu.SemaphoreType.DMA,
            pltpu.SemaphoreType.DMA,
            pltpu.SemaphoreType.DMA,
            pltpu.SemaphoreType.DMA,
        ],
    )
    out = sc(x.astype(jnp.int32), t2)
    return out.reshape(B, 1)


# speedup vs baseline: 1.1515x; 1.0210x over previous
import jax
import jax.numpy as jnp
from jax import lax
from jax.experimental import pallas as pl
from jax.experimental.pallas import tpu as pltpu
from jax.experimental.pallas import tpu_sc as plsc

B = 4096
G = 4
L = 50
D = 128
V = 10000
NW = 32
ROWS_PER_W = B // NW


def _tc_project(b_ref, w_ref, table_ref, out_ref):
    out_ref[...] = lax.dot_general(
        w_ref[...], table_ref[...],
        (((1,), (1,)), ((), ())),
        preferred_element_type=jnp.float32,
    ) * (1.0 / L) + b_ref[0] * (1.0 / (G * L))


def _sc_pool(x_hbm, t2_hbm, out_hbm, idx_v, t2_v, out_v, sem_i0, sem_t0):
    wid = lax.axis_index("s") * 2 + lax.axis_index("c")
    base_row = wid * ROWS_PER_W
    cp_idx = pltpu.async_copy(x_hbm.at[pl.ds(base_row, ROWS_PER_W)], idx_v,
                              sem_i0)
    cp_t2 = pltpu.async_copy(t2_hbm, t2_v, sem_t0)
    lanes = lax.iota(jnp.int32, 16)
    cp_idx.wait()
    cp_t2.wait()

    def rg_body(rg, _):
        rv = lanes + rg * 16
        acc = jnp.zeros((16,), jnp.float32)
        for g in range(G):
            gv = jnp.full((16,), g, jnp.int32)
            lv = lanes
            for l0 in range(L):
                iv = plsc.load_gather(idx_v, [rv, gv, lv])
                acc = acc + plsc.load_gather(t2_v, [gv, iv])
                lvn = lv + 1
                lv = jnp.where(lvn == L, 0, lvn)
        out_v[pl.ds(rg * 16, 16)] = acc
        return 0

    lax.fori_loop(0, ROWS_PER_W // 16, rg_body, 0)
    pltpu.sync_copy(out_v, out_hbm.at[pl.ds(base_row, ROWS_PER_W)])


def kernel(x, table, W, b):
    w2 = W.reshape(G, D)
    t2 = pl.pallas_call(
        _tc_project,
        in_specs=[
            pl.BlockSpec(memory_space=pltpu.SMEM),
            pl.BlockSpec(memory_space=pltpu.VMEM),
            pl.BlockSpec(memory_space=pltpu.VMEM),
        ],
        out_shape=jax.ShapeDtypeStruct((G, V), jnp.float32),
    )(b, w2, table)

    sc = pl.kernel(
        _sc_pool,
        out_type=jax.ShapeDtypeStruct((B,), jnp.float32),
        mesh=plsc.VectorSubcoreMesh(core_axis_name="c", subcore_axis_name="s"),
        compiler_params=pltpu.CompilerParams(needs_layout_passes=False,
                                             use_tc_tiling_on_sc=True),
        scratch_types=[
            pltpu.VMEM((ROWS_PER_W, G, L), jnp.int32),
            pltpu.VMEM((G, V), jnp.float32),
            pltpu.VMEM((ROWS_PER_W,), jnp.float32),
            pltpu.SemaphoreType.DMA,
            pltpu.SemaphoreType.DMA,
        ],
    )
    out = sc(x.astype(jnp.int32), t2)
    return out.reshape(B, 1)
